# Initial kernel scaffold; baseline (speedup 1.0000x reference)
import jax
import jax.numpy as jnp
from jax.experimental import pallas as pl


def kernel(cls_logits, reg_deltas):
    raise NotImplementedError("write your pallas kernel here")


# speedup vs baseline: 1.2169x; 1.2169x over previous
import jax
import jax.numpy as jnp
from jax import lax
from jax.experimental import pallas as pl
from jax.experimental.pallas import tpu as pltpu

_BS = 8
_FMAP_H, _FMAP_W = 50, 50
_NA = 8
_IMG_H, _IMG_W = 800, 800
_N_ANCH = _FMAP_H * _FMAP_W * _NA
_NMS_T = 0.7
_K_PRE = 1000
_K_POST = 300
_ROWS = 160
_NPAD = _ROWS * 128
_KROWS = 8
_KC = _KROWS * 128

_NEG_INF = float("-inf")


def _anchor_planes():
    scales = jnp.array([64.0, 128.0, 256.0, 512.0], dtype=jnp.float32)
    ratios = jnp.array([0.5, 1.0], dtype=jnp.float32)
    ws = (scales[None, :] / jnp.sqrt(ratios)[:, None]).reshape(-1)
    hs = (scales[None, :] * jnp.sqrt(ratios)[:, None]).reshape(-1)
    sx = (jnp.arange(_FMAP_W, dtype=jnp.float32) + 0.5) * (_IMG_W / _FMAP_W)
    sy = (jnp.arange(_FMAP_H, dtype=jnp.float32) + 0.5) * (_IMG_H / _FMAP_H)
    yy, xx = jnp.meshgrid(sy, sx, indexing="ij")
    cx = xx.reshape(-1)
    cy = yy.reshape(-1)
    aw = jnp.broadcast_to(ws[None, :], (_FMAP_H * _FMAP_W, _NA)).reshape(-1)
    ah = jnp.broadcast_to(hs[None, :], (_FMAP_H * _FMAP_W, _NA)).reshape(-1)
    acx = jnp.broadcast_to(cx[:, None], (_FMAP_H * _FMAP_W, _NA)).reshape(-1)
    acy = jnp.broadcast_to(cy[:, None], (_FMAP_H * _FMAP_W, _NA)).reshape(-1)

    def pad(v, fill):
        return jnp.concatenate(
            [v, jnp.full((_NPAD - _N_ANCH,), fill, jnp.float32)]
        ).reshape(_ROWS, 128)

    return pad(aw, 1.0), pad(ah, 1.0), pad(acx, 0.0), pad(acy, 0.0)


def _decode_body(lg_ref, dx_ref, dy_ref, dw_ref, dh_ref,
                 aw_ref, ah_ref, acx_ref, acy_ref,
                 s_ref, x1_ref, y1_ref, x2_ref, y2_ref):
    lg = lg_ref[0]
    s_ref[0] = jax.nn.sigmoid(lg)
    w = aw_ref[...]
    h = ah_ref[...]
    cx = acx_ref[...]
    cy = acy_ref[...]
    dx = dx_ref[0]
    dy = dy_ref[0]
    dw = jnp.clip(dw_ref[0], -4.0, 4.0)
    dh = jnp.clip(dh_ref[0], -4.0, 4.0)
    pcx = dx * w + cx
    pcy = dy * h + cy
    pw = jnp.exp(dw) * w
    ph = jnp.exp(dh) * h
    x1_ref[0] = jnp.clip(pcx - 0.5 * pw, 0.0, float(_IMG_W))
    y1_ref[0] = jnp.clip(pcy - 0.5 * ph, 0.0, float(_IMG_H))
    x2_ref[0] = jnp.clip(pcx + 0.5 * pw, 0.0, float(_IMG_W))
    y2_ref[0] = jnp.clip(pcy + 0.5 * ph, 0.0, float(_IMG_H))


def _decode(logits_p, dx_p, dy_p, dw_p, dh_p):
    aw, ah, acx, acy = _anchor_planes()
    bspec = pl.BlockSpec((1, _ROWS, 128), lambda b: (b, 0, 0))
    aspec = pl.BlockSpec((_ROWS, 128), lambda b: (0, 0))
    out = jax.ShapeDtypeStruct((_BS, _ROWS, 128), jnp.float32)
    return pl.pallas_call(
        _decode_body,
        grid=(_BS,),
        in_specs=[bspec] * 5 + [aspec] * 4,
        out_specs=[bspec] * 5,
        out_shape=[out] * 5,
    )(logits_p, dx_p, dy_p, dw_p, dh_p, aw, ah, acx, acy)


def _nms_body(s_ref, x1_ref, y1_ref, x2_ref, y2_ref,
              kx1_ref, ky1_ref, kx2_ref, ky2_ref, ks_ref):
    s = s_ref[0]
    x1 = x1_ref[0]
    y1 = y1_ref[0]
    x2 = x2_ref[0]
    y2 = y2_ref[0]
    area = (x2 - x1) * (y2 - y1)
    rowi = lax.broadcasted_iota(jnp.int32, (_KROWS, 128), 0)
    lane = lax.broadcasted_iota(jnp.int32, (_KROWS, 128), 1)
    lin = rowi * 128 + lane
    zeros = jnp.zeros((_KROWS, 128), jnp.float32)

    def body(i, carry):
        cur, kx1, ky1, kx2, ky2, ks = carry
        m = jnp.max(cur)
        j = jnp.min(jnp.where(cur == m, lin, jnp.int32(1 << 30)))
        onehot = lin == j
        vf = jnp.where(m == _NEG_INF, 0.0, 1.0)
        bx1 = jnp.sum(jnp.where(onehot, x1, 0.0))
        by1 = jnp.sum(jnp.where(onehot, y1, 0.0))
        bx2 = jnp.sum(jnp.where(onehot, x2, 0.0))
        by2 = jnp.sum(jnp.where(onehot, y2, 0.0))
        bs = jnp.sum(jnp.where(onehot, s, 0.0))
        aj = jnp.sum(jnp.where(onehot, area, 0.0))
        wx = jnp.maximum(jnp.minimum(x2, bx2) - jnp.maximum(x1, bx1), 0.0)
        wy = jnp.maximum(jnp.minimum(y2, by2) - jnp.maximum(y1, by1), 0.0)
        inter = wx * wy
        iou = inter / (area + aj - inter + 1e-9)
        cur = jnp.where(iou < _NMS_T, cur, _NEG_INF)
        sloti = lin == i
        kx1 = jnp.where(sloti, bx1 * vf, kx1)
        ky1 = jnp.where(sloti, by1 * vf, ky1)
        kx2 = jnp.where(sloti, bx2 * vf, kx2)
        ky2 = jnp.where(sloti, by2 * vf, ky2)
        ks = jnp.where(sloti, bs * vf, ks)
        return (cur, kx1, ky1, kx2, ky2, ks)

    _, kx1, ky1, kx2, ky2, ks = lax.fori_loop(
        0, _K_POST, body, (s, zeros, zeros, zeros, zeros, zeros))
    kx1_ref[0] = kx1
    ky1_ref[0] = ky1
    kx2_ref[0] = kx2
    ky2_ref[0] = ky2
    ks_ref[0] = ks


def _nms(ts, tx1, ty1, tx2, ty2):
    bspec = pl.BlockSpec((1, _KROWS, 128), lambda b: (b, 0, 0))
    out = jax.ShapeDtypeStruct((_BS, _KROWS, 128), jnp.float32)
    return pl.pallas_call(
        _nms_body,
        grid=(_BS,),
        in_specs=[bspec] * 5,
        out_specs=[bspec] * 5,
        out_shape=[out] * 5,
    )(ts, tx1, ty1, tx2, ty2)


def kernel(cls_logits, reg_deltas):
    cls_logits = lax.stop_gradient(cls_logits)
    reg_deltas = lax.stop_gradient(reg_deltas)
    lg = cls_logits.reshape(_BS, _N_ANCH)
    pad1 = jnp.full((_BS, _NPAD - _N_ANCH), _NEG_INF, jnp.float32)
    lg_p = jnp.concatenate([lg, pad1], axis=1).reshape(_BS, _ROWS, 128)
    d = reg_deltas.reshape(_BS, _N_ANCH, 4)
    pad0 = jnp.zeros((_BS, _NPAD - _N_ANCH), jnp.float32)

    def padp(v):
        return jnp.concatenate([v, pad0], axis=1).reshape(_BS, _ROWS, 128)

    dxp, dyp, dwp, dhp = (padp(d[:, :, k]) for k in range(4))
    s, x1, y1, x2, y2 = _decode(lg_p, dxp, dyp, dwp, dhp)

    sf = s.reshape(_BS, _NPAD)[:, :_N_ANCH]
    top_s, ids = lax.top_k(sf, _K_PRE)

    def gather(v):
        g = jnp.take_along_axis(v.reshape(_BS, _NPAD), ids, axis=1)
        return jnp.concatenate(
            [g, jnp.zeros((_BS, _KC - _K_PRE), jnp.float32)], axis=1
        ).reshape(_BS, _KROWS, 128)

    ts = jnp.concatenate(
        [top_s, jnp.full((_BS, _KC - _K_PRE), _NEG_INF, jnp.float32)], axis=1
    ).reshape(_BS, _KROWS, 128)
    kx1, ky1, kx2, ky2, ks = _nms(ts, gather(x1), gather(y1),
                                  gather(x2), gather(y2))

    def flat(v):
        return v.reshape(_BS, _KC)[:, :_K_POST]

    return jnp.stack([flat(kx1), flat(ky1), flat(kx2), flat(ky2), flat(ks)],
                     axis=-1)


# speedup vs baseline: 4.3505x; 3.5750x over previous
import jax
import jax.numpy as jnp
from jax import lax
from jax.experimental import pallas as pl
from jax.experimental.pallas import tpu as pltpu

_BS = 8
_FMAP_H, _FMAP_W = 50, 50
_NA = 8
_IMG_H, _IMG_W = 800, 800
_N_ANCH = _FMAP_H * _FMAP_W * _NA
_NMS_T = 0.7
_K_PRE = 1000
_K_POST = 300
_ROWS = 160
_NPAD = _ROWS * 128
_KROWS = 8
_KC = _KROWS * 128

_NEG_INF = float("-inf")


def _anchor_planes():
    scales = jnp.array([64.0, 128.0, 256.0, 512.0], dtype=jnp.float32)
    ratios = jnp.array([0.5, 1.0], dtype=jnp.float32)
    ws = (scales[None, :] / jnp.sqrt(ratios)[:, None]).reshape(-1)
    hs = (scales[None, :] * jnp.sqrt(ratios)[:, None]).reshape(-1)
    sx = (jnp.arange(_FMAP_W, dtype=jnp.float32) + 0.5) * (_IMG_W / _FMAP_W)
    sy = (jnp.arange(_FMAP_H, dtype=jnp.float32) + 0.5) * (_IMG_H / _FMAP_H)
    yy, xx = jnp.meshgrid(sy, sx, indexing="ij")
    cx = xx.reshape(-1)
    cy = yy.reshape(-1)
    aw = jnp.broadcast_to(ws[None, :], (_FMAP_H * _FMAP_W, _NA)).reshape(-1)
    ah = jnp.broadcast_to(hs[None, :], (_FMAP_H * _FMAP_W, _NA)).reshape(-1)
    acx = jnp.broadcast_to(cx[:, None], (_FMAP_H * _FMAP_W, _NA)).reshape(-1)
    acy = jnp.broadcast_to(cy[:, None], (_FMAP_H * _FMAP_W, _NA)).reshape(-1)

    def pad(v, fill):
        return jnp.concatenate(
            [v, jnp.full((_NPAD - _N_ANCH,), fill, jnp.float32)]
        ).reshape(_ROWS, 128)

    return pad(aw, 1.0), pad(ah, 1.0), pad(acx, 0.0), pad(acy, 0.0)


def _decode_body(lg_ref, dx_ref, dy_ref, dw_ref, dh_ref,
                 aw_ref, ah_ref, acx_ref, acy_ref,
                 s_ref, x1_ref, y1_ref, x2_ref, y2_ref):
    lg = lg_ref[0]
    s_ref[0] = jax.nn.sigmoid(lg)
    w = aw_ref[...]
    h = ah_ref[...]
    cx = acx_ref[...]
    cy = acy_ref[...]
    dx = dx_ref[0]
    dy = dy_ref[0]
    dw = jnp.clip(dw_ref[0], -4.0, 4.0)
    dh = jnp.clip(dh_ref[0], -4.0, 4.0)
    pcx = dx * w + cx
    pcy = dy * h + cy
    pw = jnp.exp(dw) * w
    ph = jnp.exp(dh) * h
    x1_ref[0] = jnp.clip(pcx - 0.5 * pw, 0.0, float(_IMG_W))
    y1_ref[0] = jnp.clip(pcy - 0.5 * ph, 0.0, float(_IMG_H))
    x2_ref[0] = jnp.clip(pcx + 0.5 * pw, 0.0, float(_IMG_W))
    y2_ref[0] = jnp.clip(pcy + 0.5 * ph, 0.0, float(_IMG_H))


def _decode(logits_p, dx_p, dy_p, dw_p, dh_p):
    aw, ah, acx, acy = _anchor_planes()
    bspec = pl.BlockSpec((1, _ROWS, 128), lambda b: (b, 0, 0))
    aspec = pl.BlockSpec((_ROWS, 128), lambda b: (0, 0))
    out = jax.ShapeDtypeStruct((_BS, _ROWS, 128), jnp.float32)
    return pl.pallas_call(
        _decode_body,
        grid=(_BS,),
        in_specs=[bspec] * 5 + [aspec] * 4,
        out_specs=[bspec] * 5,
        out_shape=[out] * 5,
    )(logits_p, dx_p, dy_p, dw_p, dh_p, aw, ah, acx, acy)


def _nms_body(s_ref, x1_ref, y1_ref, x2_ref, y2_ref,
              kx1_ref, ky1_ref, kx2_ref, ky2_ref, ks_ref):
    s = s_ref[...]
    x1 = x1_ref[...]
    y1 = y1_ref[...]
    x2 = x2_ref[...]
    y2 = y2_ref[...]
    area = (x2 - x1) * (y2 - y1)
    shp = (_BS, _KROWS, 128)
    rowi = lax.broadcasted_iota(jnp.int32, shp, 1)
    lane = lax.broadcasted_iota(jnp.int32, shp, 2)
    lin = rowi * 128 + lane
    zeros = jnp.zeros(shp, jnp.float32)

    def red(v, kind):
        return kind(kind(v, axis=2, keepdims=True), axis=1, keepdims=True)

    def body(i, carry):
        cur, kx1, ky1, kx2, ky2, ks = carry
        m = red(cur, jnp.max)
        j = red(jnp.where(cur == m, lin, jnp.int32(1 << 30)), jnp.min)
        onehot = lin == j
        vf = jnp.where(m == _NEG_INF, 0.0, 1.0)
        bx1 = red(jnp.where(onehot, x1, 0.0), jnp.sum)
        by1 = red(jnp.where(onehot, y1, 0.0), jnp.sum)
        bx2 = red(jnp.where(onehot, x2, 0.0), jnp.sum)
        by2 = red(jnp.where(onehot, y2, 0.0), jnp.sum)
        bs = red(jnp.where(onehot, s, 0.0), jnp.sum)
        aj = red(jnp.where(onehot, area, 0.0), jnp.sum)
        wx = jnp.maximum(jnp.minimum(x2, bx2) - jnp.maximum(x1, bx1), 0.0)
        wy = jnp.maximum(jnp.minimum(y2, by2) - jnp.maximum(y1, by1), 0.0)
        inter = wx * wy
        iou = inter / (area + aj - inter + 1e-9)
        cur = jnp.where(iou < _NMS_T, cur, _NEG_INF)
        sloti = lin == i
        kx1 = jnp.where(sloti, bx1 * vf, kx1)
        ky1 = jnp.where(sloti, by1 * vf, ky1)
        kx2 = jnp.where(sloti, bx2 * vf, kx2)
        ky2 = jnp.where(sloti, by2 * vf, ky2)
        ks = jnp.where(sloti, bs * vf, ks)
        return (cur, kx1, ky1, kx2, ky2, ks)

    _, kx1, ky1, kx2, ky2, ks = lax.fori_loop(
        0, _K_POST, body, (s, zeros, zeros, zeros, zeros, zeros))
    kx1_ref[...] = kx1
    ky1_ref[...] = ky1
    kx2_ref[...] = kx2
    ky2_ref[...] = ky2
    ks_ref[...] = ks


def _nms(ts, tx1, ty1, tx2, ty2):
    out = jax.ShapeDtypeStruct((_BS, _KROWS, 128), jnp.float32)
    return pl.pallas_call(
        _nms_body,
        out_shape=[out] * 5,
    )(ts, tx1, ty1, tx2, ty2)


def kernel(cls_logits, reg_deltas):
    cls_logits = lax.stop_gradient(cls_logits)
    reg_deltas = lax.stop_gradient(reg_deltas)
    lg = cls_logits.reshape(_BS, _N_ANCH)
    pad1 = jnp.full((_BS, _NPAD - _N_ANCH), _NEG_INF, jnp.float32)
    lg_p = jnp.concatenate([lg, pad1], axis=1).reshape(_BS, _ROWS, 128)
    d = reg_deltas.reshape(_BS, _N_ANCH, 4)
    pad0 = jnp.zeros((_BS, _NPAD - _N_ANCH), jnp.float32)

    def padp(v):
        return jnp.concatenate([v, pad0], axis=1).reshape(_BS, _ROWS, 128)

    dxp, dyp, dwp, dhp = (padp(d[:, :, k]) for k in range(4))
    s, x1, y1, x2, y2 = _decode(lg_p, dxp, dyp, dwp, dhp)

    sf = s.reshape(_BS, _NPAD)[:, :_N_ANCH]
    top_s, ids = lax.top_k(sf, _K_PRE)

    def gather(v):
        g = jnp.take_along_axis(v.reshape(_BS, _NPAD), ids, axis=1)
        return jnp.concatenate(
            [g, jnp.zeros((_BS, _KC - _K_PRE), jnp.float32)], axis=1
        ).reshape(_BS, _KROWS, 128)

    ts = jnp.concatenate(
        [top_s, jnp.full((_BS, _KC - _K_PRE), _NEG_INF, jnp.float32)], axis=1
    ).reshape(_BS, _KROWS, 128)
    kx1, ky1, kx2, ky2, ks = _nms(ts, gather(x1), gather(y1),
                                  gather(x2), gather(y2))

    def flat(v):
        return v.reshape(_BS, _KC)[:, :_K_POST]

    return jnp.stack([flat(kx1), flat(ky1), flat(kx2), flat(ky2), flat(ks)],
                     axis=-1)


# speedup vs baseline: 4.3517x; 1.0003x over previous
import jax
import jax.numpy as jnp
from jax import lax
from jax.experimental import pallas as pl
from jax.experimental.pallas import tpu as pltpu

_BS = 8
_FMAP_H, _FMAP_W = 50, 50
_NA = 8
_IMG_H, _IMG_W = 800, 800
_N_ANCH = _FMAP_H * _FMAP_W * _NA
_NMS_T = 0.7
_K_PRE = 1000
_K_POST = 300
_ROWS = 160
_NPAD = _ROWS * 128
_KROWS = 8
_KC = _KROWS * 128

_NEG_INF = float("-inf")


def _anchor_planes():
    scales = jnp.array([64.0, 128.0, 256.0, 512.0], dtype=jnp.float32)
    ratios = jnp.array([0.5, 1.0], dtype=jnp.float32)
    ws = (scales[None, :] / jnp.sqrt(ratios)[:, None]).reshape(-1)
    hs = (scales[None, :] * jnp.sqrt(ratios)[:, None]).reshape(-1)
    sx = (jnp.arange(_FMAP_W, dtype=jnp.float32) + 0.5) * (_IMG_W / _FMAP_W)
    sy = (jnp.arange(_FMAP_H, dtype=jnp.float32) + 0.5) * (_IMG_H / _FMAP_H)
    yy, xx = jnp.meshgrid(sy, sx, indexing="ij")
    cx = xx.reshape(-1)
    cy = yy.reshape(-1)
    aw = jnp.broadcast_to(ws[None, :], (_FMAP_H * _FMAP_W, _NA)).reshape(-1)
    ah = jnp.broadcast_to(hs[None, :], (_FMAP_H * _FMAP_W, _NA)).reshape(-1)
    acx = jnp.broadcast_to(cx[:, None], (_FMAP_H * _FMAP_W, _NA)).reshape(-1)
    acy = jnp.broadcast_to(cy[:, None], (_FMAP_H * _FMAP_W, _NA)).reshape(-1)

    def pad(v, fill):
        return jnp.concatenate(
            [v, jnp.full((_NPAD - _N_ANCH,), fill, jnp.float32)]
        ).reshape(_ROWS, 128)

    return pad(aw, 1.0), pad(ah, 1.0), pad(acx, 0.0), pad(acy, 0.0)


def _decode_body(lg_ref, dx_ref, dy_ref, dw_ref, dh_ref,
                 aw_ref, ah_ref, acx_ref, acy_ref,
                 s_ref, x1_ref, y1_ref, x2_ref, y2_ref):
    lg = lg_ref[0]
    s_ref[0] = jax.nn.sigmoid(lg)
    w = aw_ref[...]
    h = ah_ref[...]
    cx = acx_ref[...]
    cy = acy_ref[...]
    dx = dx_ref[0]
    dy = dy_ref[0]
    dw = jnp.clip(dw_ref[0], -4.0, 4.0)
    dh = jnp.clip(dh_ref[0], -4.0, 4.0)
    pcx = dx * w + cx
    pcy = dy * h + cy
    pw = jnp.exp(dw) * w
    ph = jnp.exp(dh) * h
    x1_ref[0] = jnp.clip(pcx - 0.5 * pw, 0.0, float(_IMG_W))
    y1_ref[0] = jnp.clip(pcy - 0.5 * ph, 0.0, float(_IMG_H))
    x2_ref[0] = jnp.clip(pcx + 0.5 * pw, 0.0, float(_IMG_W))
    y2_ref[0] = jnp.clip(pcy + 0.5 * ph, 0.0, float(_IMG_H))


def _decode(logits_p, dx_p, dy_p, dw_p, dh_p):
    aw, ah, acx, acy = _anchor_planes()
    bspec = pl.BlockSpec((1, _ROWS, 128), lambda b: (b, 0, 0))
    aspec = pl.BlockSpec((_ROWS, 128), lambda b: (0, 0))
    out = jax.ShapeDtypeStruct((_BS, _ROWS, 128), jnp.float32)
    return pl.pallas_call(
        _decode_body,
        grid=(_BS,),
        in_specs=[bspec] * 5 + [aspec] * 4,
        out_specs=[bspec] * 5,
        out_shape=[out] * 5,
    )(logits_p, dx_p, dy_p, dw_p, dh_p, aw, ah, acx, acy)


def _nms_body(s_ref, x1_ref, y1_ref, x2_ref, y2_ref,
              kx1_ref, ky1_ref, kx2_ref, ky2_ref, ks_ref):
    s = s_ref[...]
    x1 = x1_ref[...]
    y1 = y1_ref[...]
    x2 = x2_ref[...]
    y2 = y2_ref[...]
    area = (x2 - x1) * (y2 - y1)
    shp = (_BS, _KROWS, 128)
    rowi = lax.broadcasted_iota(jnp.int32, shp, 1)
    lane = lax.broadcasted_iota(jnp.int32, shp, 2)
    lin = rowi * 128 + lane
    zeros = jnp.zeros(shp, jnp.float32)

    def red(v, kind):
        return kind(kind(v, axis=2, keepdims=True), axis=1, keepdims=True)

    lane3 = lax.broadcasted_iota(jnp.int32, (_BS, 1, 128), 2)
    zrow = jnp.zeros((_BS, 1, 128), jnp.float32)
    cur = s
    for r in range((_K_POST + 127) // 128):
        lo = r * 128
        hi = min(_K_POST, lo + 128)

        def body(i, carry, lo=lo):
            cur, a1, a2, a3, a4, a5 = carry
            m = red(cur, jnp.max)
            j = red(jnp.where(cur == m, lin, jnp.int32(1 << 30)), jnp.min)
            onehot = lin == j
            vf = jnp.where(m == _NEG_INF, 0.0, 1.0)
            bx1 = red(jnp.where(onehot, x1, 0.0), jnp.sum)
            by1 = red(jnp.where(onehot, y1, 0.0), jnp.sum)
            bx2 = red(jnp.where(onehot, x2, 0.0), jnp.sum)
            by2 = red(jnp.where(onehot, y2, 0.0), jnp.sum)
            bs = red(jnp.where(onehot, s, 0.0), jnp.sum)
            aj = red(jnp.where(onehot, area, 0.0), jnp.sum)
            wx = jnp.maximum(jnp.minimum(x2, bx2) - jnp.maximum(x1, bx1), 0.0)
            wy = jnp.maximum(jnp.minimum(y2, by2) - jnp.maximum(y1, by1), 0.0)
            inter = wx * wy
            iou = inter / (area + aj - inter + 1e-9)
            cur = jnp.where(iou < _NMS_T, cur, _NEG_INF)
            sloti = lane3 == (i - lo)
            a1 = jnp.where(sloti, bx1 * vf, a1)
            a2 = jnp.where(sloti, by1 * vf, a2)
            a3 = jnp.where(sloti, bx2 * vf, a3)
            a4 = jnp.where(sloti, by2 * vf, a4)
            a5 = jnp.where(sloti, bs * vf, a5)
            return (cur, a1, a2, a3, a4, a5)

        cur, a1, a2, a3, a4, a5 = lax.fori_loop(
            lo, hi, body, (cur, zrow, zrow, zrow, zrow, zrow))
        kx1_ref[:, r:r + 1, :] = a1
        ky1_ref[:, r:r + 1, :] = a2
        kx2_ref[:, r:r + 1, :] = a3
        ky2_ref[:, r:r + 1, :] = a4
        ks_ref[:, r:r + 1, :] = a5


def _nms(ts, tx1, ty1, tx2, ty2):
    out = jax.ShapeDtypeStruct((_BS, _KROWS, 128), jnp.float32)
    return pl.pallas_call(
        _nms_body,
        out_shape=[out] * 5,
    )(ts, tx1, ty1, tx2, ty2)


def kernel(cls_logits, reg_deltas):
    cls_logits = lax.stop_gradient(cls_logits)
    reg_deltas = lax.stop_gradient(reg_deltas)
    lg = cls_logits.reshape(_BS, _N_ANCH)
    pad1 = jnp.full((_BS, _NPAD - _N_ANCH), _NEG_INF, jnp.float32)
    lg_p = jnp.concatenate([lg, pad1], axis=1).reshape(_BS, _ROWS, 128)
    d = reg_deltas.reshape(_BS, _N_ANCH, 4)
    pad0 = jnp.zeros((_BS, _NPAD - _N_ANCH), jnp.float32)

    def padp(v):
        return jnp.concatenate([v, pad0], axis=1).reshape(_BS, _ROWS, 128)

    dxp, dyp, dwp, dhp = (padp(d[:, :, k]) for k in range(4))
    s, x1, y1, x2, y2 = _decode(lg_p, dxp, dyp, dwp, dhp)

    sf = s.reshape(_BS, _NPAD)[:, :_N_ANCH]
    top_s, ids = lax.top_k(sf, _K_PRE)

    def gather(v):
        g = jnp.take_along_axis(v.reshape(_BS, _NPAD), ids, axis=1)
        return jnp.concatenate(
            [g, jnp.zeros((_BS, _KC - _K_PRE), jnp.float32)], axis=1
        ).reshape(_BS, _KROWS, 128)

    ts = jnp.concatenate(
        [top_s, jnp.full((_BS, _KC - _K_PRE), _NEG_INF, jnp.float32)], axis=1
    ).reshape(_BS, _KROWS, 128)
    kx1, ky1, kx2, ky2, ks = _nms(ts, gather(x1), gather(y1),
                                  gather(x2), gather(y2))

    def flat(v):
        return v.reshape(_BS, _KC)[:, :_K_POST]

    return jnp.stack([flat(kx1), flat(ky1), flat(kx2), flat(ky2), flat(ks)],
                     axis=-1)


# speedup vs baseline: 4.3602x; 1.0019x over previous
import jax
import jax.numpy as jnp
from jax import lax
from jax.experimental import pallas as pl
from jax.experimental.pallas import tpu as pltpu

_BS = 8
_FMAP_H, _FMAP_W = 50, 50
_NA = 8
_IMG_H, _IMG_W = 800, 800
_N_ANCH = _FMAP_H * _FMAP_W * _NA
_NMS_T = 0.7
_K_PRE = 1000
_K_POST = 300
_ROWS = 160
_NPAD = _ROWS * 128
_KROWS = 8
_KC = _KROWS * 128

_NEG_INF = float("-inf")


def _anchor_planes():
    scales = jnp.array([64.0, 128.0, 256.0, 512.0], dtype=jnp.float32)
    ratios = jnp.array([0.5, 1.0], dtype=jnp.float32)
    ws = (scales[None, :] / jnp.sqrt(ratios)[:, None]).reshape(-1)
    hs = (scales[None, :] * jnp.sqrt(ratios)[:, None]).reshape(-1)
    sx = (jnp.arange(_FMAP_W, dtype=jnp.float32) + 0.5) * (_IMG_W / _FMAP_W)
    sy = (jnp.arange(_FMAP_H, dtype=jnp.float32) + 0.5) * (_IMG_H / _FMAP_H)
    yy, xx = jnp.meshgrid(sy, sx, indexing="ij")
    cx = xx.reshape(-1)
    cy = yy.reshape(-1)
    aw = jnp.broadcast_to(ws[None, :], (_FMAP_H * _FMAP_W, _NA)).reshape(-1)
    ah = jnp.broadcast_to(hs[None, :], (_FMAP_H * _FMAP_W, _NA)).reshape(-1)
    acx = jnp.broadcast_to(cx[:, None], (_FMAP_H * _FMAP_W, _NA)).reshape(-1)
    acy = jnp.broadcast_to(cy[:, None], (_FMAP_H * _FMAP_W, _NA)).reshape(-1)

    def pad(v, fill):
        return jnp.concatenate(
            [v, jnp.full((_NPAD - _N_ANCH,), fill, jnp.float32)]
        ).reshape(_ROWS, 128)

    return pad(aw, 1.0), pad(ah, 1.0), pad(acx, 0.0), pad(acy, 0.0)


def _decode_body(lg_ref, dx_ref, dy_ref, dw_ref, dh_ref,
                 aw_ref, ah_ref, acx_ref, acy_ref,
                 s_ref, x1_ref, y1_ref, x2_ref, y2_ref):
    lg = lg_ref[0]
    s_ref[0] = jax.nn.sigmoid(lg)
    w = aw_ref[...]
    h = ah_ref[...]
    cx = acx_ref[...]
    cy = acy_ref[...]
    dx = dx_ref[0]
    dy = dy_ref[0]
    dw = jnp.clip(dw_ref[0], -4.0, 4.0)
    dh = jnp.clip(dh_ref[0], -4.0, 4.0)
    pcx = dx * w + cx
    pcy = dy * h + cy
    pw = jnp.exp(dw) * w
    ph = jnp.exp(dh) * h
    x1_ref[0] = jnp.clip(pcx - 0.5 * pw, 0.0, float(_IMG_W))
    y1_ref[0] = jnp.clip(pcy - 0.5 * ph, 0.0, float(_IMG_H))
    x2_ref[0] = jnp.clip(pcx + 0.5 * pw, 0.0, float(_IMG_W))
    y2_ref[0] = jnp.clip(pcy + 0.5 * ph, 0.0, float(_IMG_H))


def _decode(logits_p, dx_p, dy_p, dw_p, dh_p):
    aw, ah, acx, acy = _anchor_planes()
    bspec = pl.BlockSpec((1, _ROWS, 128), lambda b: (b, 0, 0))
    aspec = pl.BlockSpec((_ROWS, 128), lambda b: (0, 0))
    out = jax.ShapeDtypeStruct((_BS, _ROWS, 128), jnp.float32)
    return pl.pallas_call(
        _decode_body,
        grid=(_BS,),
        in_specs=[bspec] * 5 + [aspec] * 4,
        out_specs=[bspec] * 5,
        out_shape=[out] * 5,
    )(logits_p, dx_p, dy_p, dw_p, dh_p, aw, ah, acx, acy)


def _nms_body(s_ref, x1_ref, y1_ref, x2_ref, y2_ref,
              kx1_ref, ky1_ref, kx2_ref, ky2_ref, ks_ref):
    s = s_ref[...]
    x1 = x1_ref[...]
    y1 = y1_ref[...]
    x2 = x2_ref[...]
    y2 = y2_ref[...]
    area = (x2 - x1) * (y2 - y1)
    shp = (_BS, _KROWS, 128)
    rowi = lax.broadcasted_iota(jnp.int32, shp, 1)
    lane = lax.broadcasted_iota(jnp.int32, shp, 2)
    lin = rowi * 128 + lane

    def red(v, kind):
        return kind(kind(v, axis=2, keepdims=True), axis=1, keepdims=True)

    lane3 = lax.broadcasted_iota(jnp.int32, (_BS, 1, 128), 2)
    zrow = jnp.zeros((_BS, 1, 128), jnp.float32)
    cur = s
    for r in range((_K_POST + 127) // 128):
        lo = r * 128
        hi = min(_K_POST, lo + 128)

        def body(i, carry, lo=lo):
            cur, a1, a2, a3, a4, a5 = carry
            m = red(cur, jnp.max)
            j = red(jnp.where(cur == m, lin, jnp.int32(1 << 30)), jnp.min)
            onehot = lin == j
            vf = jnp.where(m == _NEG_INF, 0.0, 1.0)
            bx1 = red(jnp.where(onehot, x1, 0.0), jnp.sum)
            by1 = red(jnp.where(onehot, y1, 0.0), jnp.sum)
            bx2 = red(jnp.where(onehot, x2, 0.0), jnp.sum)
            by2 = red(jnp.where(onehot, y2, 0.0), jnp.sum)
            bs = red(jnp.where(onehot, s, 0.0), jnp.sum)
            aj = red(jnp.where(onehot, area, 0.0), jnp.sum)
            wx = jnp.maximum(jnp.minimum(x2, bx2) - jnp.maximum(x1, bx1), 0.0)
            wy = jnp.maximum(jnp.minimum(y2, by2) - jnp.maximum(y1, by1), 0.0)
            inter = wx * wy
            iou = inter / (area + aj - inter + 1e-9)
            cur = jnp.where(iou < _NMS_T, cur, _NEG_INF)
            sloti = lane3 == (i - lo)
            a1 = jnp.where(sloti, bx1 * vf, a1)
            a2 = jnp.where(sloti, by1 * vf, a2)
            a3 = jnp.where(sloti, bx2 * vf, a3)
            a4 = jnp.where(sloti, by2 * vf, a4)
            a5 = jnp.where(sloti, bs * vf, a5)
            return (cur, a1, a2, a3, a4, a5)

        cur, a1, a2, a3, a4, a5 = lax.fori_loop(
            lo, hi, body, (cur, zrow, zrow, zrow, zrow, zrow))
        kx1_ref[:, r:r + 1, :] = a1
        ky1_ref[:, r:r + 1, :] = a2
        kx2_ref[:, r:r + 1, :] = a3
        ky2_ref[:, r:r + 1, :] = a4
        ks_ref[:, r:r + 1, :] = a5


def _nms(ts, tx1, ty1, tx2, ty2):
    out = jax.ShapeDtypeStruct((_BS, _KROWS, 128), jnp.float32)
    return pl.pallas_call(
        _nms_body,
        out_shape=[out] * 5,
    )(ts, tx1, ty1, tx2, ty2)


def kernel(cls_logits, reg_deltas):
    cls_logits = lax.stop_gradient(cls_logits)
    reg_deltas = lax.stop_gradient(reg_deltas)
    lg = cls_logits.reshape(_BS, _N_ANCH)
    pad1 = jnp.full((_BS, _NPAD - _N_ANCH), _NEG_INF, jnp.float32)
    lg_p = jnp.concatenate([lg, pad1], axis=1).reshape(_BS, _ROWS, 128)
    d = reg_deltas.reshape(_BS, _N_ANCH, 4)
    pad0 = jnp.zeros((_BS, _NPAD - _N_ANCH), jnp.float32)

    def padp(v):
        return jnp.concatenate([v, pad0], axis=1).reshape(_BS, _ROWS, 128)

    dxp, dyp, dwp, dhp = (padp(d[:, :, k]) for k in range(4))
    s, x1, y1, x2, y2 = _decode(lg_p, dxp, dyp, dwp, dhp)

    sf = s.reshape(_BS, _NPAD)[:, :_N_ANCH]
    top_s, ids = lax.top_k(sf, _K_PRE)

    def gather(v):
        g = jnp.take_along_axis(v.reshape(_BS, _NPAD), ids, axis=1)
        return jnp.concatenate(
            [g, jnp.zeros((_BS, _KC - _K_PRE), jnp.float32)], axis=1
        ).reshape(_BS, _KROWS, 128)

    ts = jnp.concatenate(
        [top_s, jnp.full((_BS, _KC - _K_PRE), _NEG_INF, jnp.float32)], axis=1
    ).reshape(_BS, _KROWS, 128)
    kx1, ky1, kx2, ky2, ks = _nms(ts, gather(x1), gather(y1),
                                  gather(x2), gather(y2))

    def flat(v):
        return v.reshape(_BS, _KC)[:, :_K_POST]

    return jnp.stack([flat(kx1), flat(ky1), flat(kx2), flat(ky2), flat(ks)],
                     axis=-1)
